# baseline (speedup 1.0000x reference)
import jax
import jax.numpy as jnp
from jax import lax
from jax.experimental import pallas as pl
from jax.experimental.pallas import tpu as pltpu

_LN_EPS = 1e-5
_TILE_M = 128


def _fused_prenorm_linear_kernel(x_ref, g_ref, be_ref, w_ref, bias_ref, o_ref):
    x = x_ref[...]
    mean = jnp.mean(x, axis=-1, keepdims=True)
    c = x - mean
    var = jnp.mean(c * c, axis=-1, keepdims=True)
    xn = c * lax.rsqrt(var + _LN_EPS)
    xn = xn * g_ref[...] + be_ref[...]
    acc = jnp.dot(xn.astype(jnp.bfloat16), w_ref[...],
                  preferred_element_type=jnp.float32)
    o_ref[...] = acc + bias_ref[...]


def kernel(x, gamma, beta, w, b):
    B, N, D = x.shape
    Dout = w.shape[1]
    rows = B * N
    x2 = x.reshape(rows, D)
    w_bf16 = w.astype(jnp.bfloat16)

    tile_m = min(_TILE_M, rows)
    grid_m = pl.cdiv(rows, tile_m)

    out = pl.pallas_call(
        _fused_prenorm_linear_kernel,
        out_shape=jax.ShapeDtypeStruct((rows, Dout), x.dtype),
        grid_spec=pltpu.PrefetchScalarGridSpec(
            num_scalar_prefetch=0,
            grid=(grid_m,),
            in_specs=[
                pl.BlockSpec((tile_m, D), lambda i: (i, 0)),
                pl.BlockSpec((1, D), lambda i: (0, 0)),
                pl.BlockSpec((1, D), lambda i: (0, 0)),
                pl.BlockSpec((D, Dout), lambda i: (0, 0)),
                pl.BlockSpec((1, Dout), lambda i: (0, 0)),
            ],
            out_specs=pl.BlockSpec((tile_m, Dout), lambda i: (i, 0)),
        ),
        compiler_params=pltpu.CompilerParams(
            dimension_semantics=("parallel",),
            vmem_limit_bytes=58 * 1024 * 1024),
    )(x2, gamma.reshape(1, D), beta.reshape(1, D), w_bf16, b.reshape(1, Dout))
    return out.reshape(B, N, Dout)


# speedup vs baseline: 1.1580x; 1.1580x over previous
import jax
import jax.numpy as jnp
from jax import lax
from jax.experimental import pallas as pl
from jax.experimental.pallas import tpu as pltpu

_LN_EPS = 1e-5
_TILE_M = 512


def _fused_prenorm_linear_kernel(x_ref, g_ref, be_ref, w_ref, bias_ref, o_ref):
    x = x_ref[...]
    mean = jnp.mean(x, axis=-1, keepdims=True)
    c = x - mean
    var = jnp.mean(c * c, axis=-1, keepdims=True)
    xn = c * lax.rsqrt(var + _LN_EPS)
    xn = xn * g_ref[...] + be_ref[...]
    acc = jnp.dot(xn.astype(jnp.bfloat16), w_ref[...],
                  preferred_element_type=jnp.float32)
    o_ref[...] = acc + bias_ref[...]


def kernel(x, gamma, beta, w, b):
    B, N, D = x.shape
    Dout = w.shape[1]
    rows = B * N
    x2 = x.reshape(rows, D)
    w_bf16 = w.astype(jnp.bfloat16)

    tile_m = min(_TILE_M, rows)
    grid_m = pl.cdiv(rows, tile_m)

    out = pl.pallas_call(
        _fused_prenorm_linear_kernel,
        out_shape=jax.ShapeDtypeStruct((rows, Dout), x.dtype),
        grid_spec=pltpu.PrefetchScalarGridSpec(
            num_scalar_prefetch=0,
            grid=(grid_m,),
            in_specs=[
                pl.BlockSpec((tile_m, D), lambda i: (i, 0)),
                pl.BlockSpec((1, D), lambda i: (0, 0)),
                pl.BlockSpec((1, D), lambda i: (0, 0)),
                pl.BlockSpec((D, Dout), lambda i: (0, 0)),
                pl.BlockSpec((1, Dout), lambda i: (0, 0)),
            ],
            out_specs=pl.BlockSpec((tile_m, Dout), lambda i: (i, 0)),
        ),
        compiler_params=pltpu.CompilerParams(
            dimension_semantics=("parallel",),
            vmem_limit_bytes=58 * 1024 * 1024),
    )(x2, gamma.reshape(1, D), beta.reshape(1, D), w_bf16, b.reshape(1, Dout))
    return out.reshape(B, N, Dout)


# speedup vs baseline: 1.3350x; 1.1528x over previous
import jax
import jax.numpy as jnp
from jax import lax
from jax.experimental import pallas as pl
from jax.experimental.pallas import tpu as pltpu

_LN_EPS = 1e-5
_TILE_M = 512


def _fused_prenorm_linear_kernel(x_ref, g_ref, be_ref, w_ref, bias_ref, o_ref):
    x = x_ref[...]
    mean = jnp.mean(x, axis=-1, keepdims=True)
    c = x - mean
    var = jnp.mean(c * c, axis=-1, keepdims=True)
    xn = c * lax.rsqrt(var + _LN_EPS)
    xn = xn * g_ref[...] + be_ref[...]
    acc = jnp.dot(xn, w_ref[...], preferred_element_type=jnp.float32)
    o_ref[...] = acc + bias_ref[...]


def kernel(x, gamma, beta, w, b):
    B, N, D = x.shape
    Dout = w.shape[1]
    rows = B * N
    x2 = x.reshape(rows, D)

    tile_m = min(_TILE_M, rows)
    grid_m = pl.cdiv(rows, tile_m)

    out = pl.pallas_call(
        _fused_prenorm_linear_kernel,
        out_shape=jax.ShapeDtypeStruct((rows, Dout), x.dtype),
        grid_spec=pltpu.PrefetchScalarGridSpec(
            num_scalar_prefetch=0,
            grid=(grid_m,),
            in_specs=[
                pl.BlockSpec((tile_m, D), lambda i: (i, 0)),
                pl.BlockSpec((1, D), lambda i: (0, 0)),
                pl.BlockSpec((1, D), lambda i: (0, 0)),
                pl.BlockSpec((D, Dout), lambda i: (0, 0)),
                pl.BlockSpec((1, Dout), lambda i: (0, 0)),
            ],
            out_specs=pl.BlockSpec((tile_m, Dout), lambda i: (i, 0)),
        ),
        compiler_params=pltpu.CompilerParams(
            dimension_semantics=("parallel",),
            vmem_limit_bytes=58 * 1024 * 1024),
    )(x2, gamma.reshape(1, D), beta.reshape(1, D), w, b.reshape(1, Dout))
    return out.reshape(B, N, Dout)
